# baseline (speedup 1.0000x reference)
import functools

import jax
import jax.numpy as jnp
from jax import lax
from jax.experimental import pallas as pl
from jax.experimental.pallas import tpu as pltpu
from jax.experimental.pallas import tpu_sc as plsc

NUM_ROWS = 1000000
DIM = 64
BATCH = 16384

_NC = 2
_NS = 16
_NW = _NC * _NS
_B_PER_W = BATCH // _NW
_IDX_CHUNK = 128
_N_CHUNKS = _B_PER_W // _IDX_CHUNK

_SUM_BLK = 4096
_SUM_GRID = (NUM_ROWS + _SUM_BLK - 1) // _SUM_BLK
_SUM_REM = NUM_ROWS - (_SUM_GRID - 1) * _SUM_BLK
_FIN_BLK = 2048


def _sc_gather_body(table_hbm, idx_hbm, out_hbm, idx_v, rows_v, sem):
    wid = lax.axis_index("s") * _NC + lax.axis_index("c")
    base = wid * _B_PER_W
    pltpu.sync_copy(idx_hbm.at[pl.ds(base, _B_PER_W)], idx_v)
    copies = [
        pltpu.async_copy(
            table_hbm.at[idx_v.at[pl.ds(j * _IDX_CHUNK, _IDX_CHUNK)]],
            rows_v.at[pl.ds(j * _IDX_CHUNK, _IDX_CHUNK)],
            sem,
        )
        for j in range(_N_CHUNKS)
    ]
    for c in copies:
        c.wait()
    pltpu.sync_copy(rows_v, out_hbm.at[pl.ds(base, _B_PER_W)])


_sc_gather = pl.kernel(
    _sc_gather_body,
    mesh=plsc.VectorSubcoreMesh(core_axis_name="c", subcore_axis_name="s"),
    compiler_params=pltpu.CompilerParams(use_tc_tiling_on_sc=True),
    out_type=jax.ShapeDtypeStruct((BATCH, 2 * DIM), jnp.float32),
    scratch_types=[
        pltpu.VMEM((_B_PER_W,), jnp.int32),
        pltpu.VMEM((_B_PER_W, 2 * DIM), jnp.float32),
        pltpu.SemaphoreType.DMA,
    ],
)


def _prep_body(x_ref, y_ref, o_ref):
    j = pl.program_id(0)

    @pl.when(j == 0)
    def _init():
        o_ref[...] = jnp.zeros_like(o_ref)

    x = x_ref[...]
    xt = jnp.transpose(x)
    y_ref[...] = jnp.concatenate(
        [xt[: _SUM_BLK // 2], xt[_SUM_BLK // 2:]], axis=1
    )

    @pl.when(j < _SUM_GRID - 1)
    def _full():
        s = x[:, 0:128]
        for k in range(1, _SUM_BLK // 128):
            s = s + x[:, k * 128:(k + 1) * 128]
        o_ref[...] += s

    @pl.when(j == _SUM_GRID - 1)
    def _tail():
        n_full = _SUM_REM // 128
        s = x[:, 0:128]
        for k in range(1, n_full):
            s = s + x[:, k * 128:(k + 1) * 128]
        part = _SUM_REM - n_full * 128
        if part:
            tail = x[:, n_full * 128:(n_full + 1) * 128]
            lane = lax.broadcasted_iota(jnp.int32, (DIM, 128), 1)
            s = s + jnp.where(lane < part, tail, 0.0)
        o_ref[...] += s


def _prep(table_t):
    return pl.pallas_call(
        _prep_body,
        grid=(_SUM_GRID,),
        in_specs=[pl.BlockSpec((DIM, _SUM_BLK), lambda i: (0, i))],
        out_specs=[
            pl.BlockSpec((_SUM_BLK // 2, 2 * DIM), lambda i: (i, 0)),
            pl.BlockSpec((DIM, 128), lambda i: (0, 0)),
        ],
        out_shape=[
            jax.ShapeDtypeStruct((_SUM_GRID * _SUM_BLK // 2, 2 * DIM), jnp.float32),
            jax.ShapeDtypeStruct((DIM, 128), jnp.float32),
        ],
    )(table_t)


def _finalize_body(raw_ref, mean_ref, o_ref):
    x = raw_ref[...] - mean_ref[0:1, :]
    n2 = jnp.sum(x * x, axis=1, keepdims=True)
    o_ref[...] = x * lax.rsqrt(jnp.maximum(n2, 1e-24))


def _finalize(raw, mean_b):
    return pl.pallas_call(
        _finalize_body,
        grid=(BATCH // _FIN_BLK,),
        in_specs=[
            pl.BlockSpec((_FIN_BLK, DIM), lambda i: (i, 0)),
            pl.BlockSpec((8, DIM), lambda i: (0, 0)),
        ],
        out_specs=pl.BlockSpec((_FIN_BLK, DIM), lambda i: (i, 0)),
        out_shape=jax.ShapeDtypeStruct((BATCH, DIM), jnp.float32),
    )(raw, mean_b)


def kernel(indices, embedding):
    idx = indices.astype(jnp.int32)
    table2, acc = _prep(embedding.T)
    q = ((idx >> 12) << 11) | (idx & (_SUM_BLK // 2 - 1))
    raw2 = _sc_gather(table2, q)
    mean64 = jnp.sum(acc, axis=1) * (1.0 / NUM_ROWS)
    mean_b = jnp.broadcast_to(mean64[None, :], (8, DIM))
    half = (idx >> 11) & 1
    raw = jnp.where(half[:, None] == 1, raw2[:, DIM:], raw2[:, :DIM])
    return _finalize(raw, mean_b)


# speedup vs baseline: 1.1679x; 1.1679x over previous
import functools

import jax
import jax.numpy as jnp
from jax import lax
from jax.experimental import pallas as pl
from jax.experimental.pallas import tpu as pltpu
from jax.experimental.pallas import tpu_sc as plsc

NUM_ROWS = 1000000
DIM = 64
BATCH = 16384

_NC = 2
_NS = 16
_NW = _NC * _NS
_B_PER_W = BATCH // _NW
_IDX_CHUNK = 128
_N_CHUNKS = _B_PER_W // _IDX_CHUNK

_SUM_BLK = 4096
_SUM_GRID = (NUM_ROWS + _SUM_BLK - 1) // _SUM_BLK
_SUM_REM = NUM_ROWS - (_SUM_GRID - 1) * _SUM_BLK
_FIN_BLK = 2048


def _sc_gather_body(table_hbm, idx_hbm, out_hbm, idx_v, rows_v, sem):
    wid = lax.axis_index("s") * _NC + lax.axis_index("c")
    base = wid * _B_PER_W
    pltpu.sync_copy(idx_hbm.at[pl.ds(base, _B_PER_W)], idx_v)
    copies = [
        pltpu.async_copy(
            table_hbm.at[idx_v.at[pl.ds(j * _IDX_CHUNK, _IDX_CHUNK)]],
            rows_v.at[pl.ds(j * _IDX_CHUNK, _IDX_CHUNK)],
            sem,
        )
        for j in range(_N_CHUNKS)
    ]
    for c in copies:
        c.wait()
    pltpu.sync_copy(rows_v, out_hbm.at[pl.ds(base, _B_PER_W)])


_sc_gather = pl.kernel(
    _sc_gather_body,
    mesh=plsc.VectorSubcoreMesh(core_axis_name="c", subcore_axis_name="s"),
    compiler_params=pltpu.CompilerParams(use_tc_tiling_on_sc=True),
    out_type=jax.ShapeDtypeStruct((BATCH, 2 * DIM), jnp.float32),
    scratch_types=[
        pltpu.VMEM((_B_PER_W,), jnp.int32),
        pltpu.VMEM((_B_PER_W, 2 * DIM), jnp.float32),
        pltpu.SemaphoreType.DMA,
    ],
)


def _prep_body(x_ref, y_ref, o_ref):
    j = pl.program_id(0)

    @pl.when(j == 0)
    def _init():
        o_ref[...] = jnp.zeros_like(o_ref)

    x = x_ref[...]
    z = jnp.concatenate([x[:, : _SUM_BLK // 2], x[:, _SUM_BLK // 2:]], axis=0)
    ident = jnp.eye(2 * DIM, dtype=jnp.float32)
    y_ref[...] = lax.dot_general(
        z, ident, (((0,), (0,)), ((), ())), preferred_element_type=jnp.float32
    )

    @pl.when(j < _SUM_GRID - 1)
    def _full():
        s = x[:, 0:128]
        for k in range(1, _SUM_BLK // 128):
            s = s + x[:, k * 128:(k + 1) * 128]
        o_ref[...] += s

    @pl.when(j == _SUM_GRID - 1)
    def _tail():
        n_full = _SUM_REM // 128
        s = x[:, 0:128]
        for k in range(1, n_full):
            s = s + x[:, k * 128:(k + 1) * 128]
        part = _SUM_REM - n_full * 128
        if part:
            tail = x[:, n_full * 128:(n_full + 1) * 128]
            lane = lax.broadcasted_iota(jnp.int32, (DIM, 128), 1)
            s = s + jnp.where(lane < part, tail, 0.0)
        o_ref[...] += s


def _prep(table_t):
    return pl.pallas_call(
        _prep_body,
        grid=(_SUM_GRID,),
        in_specs=[pl.BlockSpec((DIM, _SUM_BLK), lambda i: (0, i))],
        out_specs=[
            pl.BlockSpec((_SUM_BLK // 2, 2 * DIM), lambda i: (i, 0)),
            pl.BlockSpec((DIM, 128), lambda i: (0, 0)),
        ],
        out_shape=[
            jax.ShapeDtypeStruct((_SUM_GRID * _SUM_BLK // 2, 2 * DIM), jnp.float32),
            jax.ShapeDtypeStruct((DIM, 128), jnp.float32),
        ],
    )(table_t)


def _finalize_body(raw_ref, mean_ref, o_ref):
    x = raw_ref[...] - mean_ref[0:1, :]
    n2 = jnp.sum(x * x, axis=1, keepdims=True)
    o_ref[...] = x * lax.rsqrt(jnp.maximum(n2, 1e-24))


def _finalize(raw, mean_b):
    return pl.pallas_call(
        _finalize_body,
        grid=(BATCH // _FIN_BLK,),
        in_specs=[
            pl.BlockSpec((_FIN_BLK, DIM), lambda i: (i, 0)),
            pl.BlockSpec((8, DIM), lambda i: (0, 0)),
        ],
        out_specs=pl.BlockSpec((_FIN_BLK, DIM), lambda i: (i, 0)),
        out_shape=jax.ShapeDtypeStruct((BATCH, DIM), jnp.float32),
    )(raw, mean_b)


def kernel(indices, embedding):
    idx = indices.astype(jnp.int32)
    table2, acc = _prep(embedding.T)
    q = ((idx >> 12) << 11) | (idx & (_SUM_BLK // 2 - 1))
    raw2 = _sc_gather(table2, q)
    mean64 = jnp.sum(acc, axis=1) * (1.0 / NUM_ROWS)
    mean_b = jnp.broadcast_to(mean64[None, :], (8, DIM))
    half = (idx >> 11) & 1
    raw = jnp.where(half[:, None] == 1, raw2[:, DIM:], raw2[:, :DIM])
    return _finalize(raw, mean_b)


# speedup vs baseline: 1.5230x; 1.3041x over previous
import functools

import jax
import jax.numpy as jnp
from jax import lax
from jax.experimental import pallas as pl
from jax.experimental.pallas import tpu as pltpu
from jax.experimental.pallas import tpu_sc as plsc

NUM_ROWS = 1000000
DIM = 64
BATCH = 16384

_NC = 2
_NS = 16
_NW = _NC * _NS
_B_PER_W = BATCH // _NW
_IDX_CHUNK = 128
_N_CHUNKS = _B_PER_W // _IDX_CHUNK

_SUM_BLK = 8192
_SUM_GRID = (NUM_ROWS + _SUM_BLK - 1) // _SUM_BLK
_SUM_REM = NUM_ROWS - (_SUM_GRID - 1) * _SUM_BLK
_FIN_BLK = 2048


def _sc_gather_body(table_hbm, idx_hbm, out_hbm, idx_v, rows_v, sem):
    wid = lax.axis_index("s") * _NC + lax.axis_index("c")
    base = wid * _B_PER_W
    pltpu.sync_copy(idx_hbm.at[pl.ds(base, _B_PER_W)], idx_v)
    copies = [
        pltpu.async_copy(
            table_hbm.at[idx_v.at[pl.ds(j * _IDX_CHUNK, _IDX_CHUNK)]],
            rows_v.at[pl.ds(j * _IDX_CHUNK, _IDX_CHUNK)],
            sem,
        )
        for j in range(_N_CHUNKS)
    ]
    for c in copies:
        c.wait()
    pltpu.sync_copy(rows_v, out_hbm.at[pl.ds(base, _B_PER_W)])


_sc_gather = pl.kernel(
    _sc_gather_body,
    mesh=plsc.VectorSubcoreMesh(core_axis_name="c", subcore_axis_name="s"),
    compiler_params=pltpu.CompilerParams(use_tc_tiling_on_sc=True),
    out_type=jax.ShapeDtypeStruct((BATCH, 2 * DIM), jnp.float32),
    scratch_types=[
        pltpu.VMEM((_B_PER_W,), jnp.int32),
        pltpu.VMEM((_B_PER_W, 2 * DIM), jnp.float32),
        pltpu.SemaphoreType.DMA,
    ],
)


def _prep_body(x_ref, y_ref, o_ref):
    j = pl.program_id(0)

    @pl.when(j == 0)
    def _init():
        o_ref[...] = jnp.zeros_like(o_ref)

    x = x_ref[...]
    z = jnp.concatenate([x[:, : _SUM_BLK // 2], x[:, _SUM_BLK // 2:]], axis=0)
    ident = jnp.eye(2 * DIM, dtype=jnp.float32)
    y_ref[...] = lax.dot_general(
        z, ident, (((0,), (0,)), ((), ())), preferred_element_type=jnp.float32
    )

    @pl.when(j < _SUM_GRID - 1)
    def _full():
        s = x[:, 0:128]
        for k in range(1, _SUM_BLK // 128):
            s = s + x[:, k * 128:(k + 1) * 128]
        o_ref[...] += s

    @pl.when(j == _SUM_GRID - 1)
    def _tail():
        n_full = _SUM_REM // 128
        s = x[:, 0:128]
        for k in range(1, n_full):
            s = s + x[:, k * 128:(k + 1) * 128]
        part = _SUM_REM - n_full * 128
        if part:
            tail = x[:, n_full * 128:(n_full + 1) * 128]
            lane = lax.broadcasted_iota(jnp.int32, (DIM, 128), 1)
            s = s + jnp.where(lane < part, tail, 0.0)
        o_ref[...] += s


def _prep(table_t):
    return pl.pallas_call(
        _prep_body,
        grid=(_SUM_GRID,),
        in_specs=[pl.BlockSpec((DIM, _SUM_BLK), lambda i: (0, i))],
        out_specs=[
            pl.BlockSpec((_SUM_BLK // 2, 2 * DIM), lambda i: (i, 0)),
            pl.BlockSpec((DIM, 128), lambda i: (0, 0)),
        ],
        out_shape=[
            jax.ShapeDtypeStruct((_SUM_GRID * _SUM_BLK // 2, 2 * DIM), jnp.float32),
            jax.ShapeDtypeStruct((DIM, 128), jnp.float32),
        ],
    )(table_t)


def _finalize_body(raw_ref, mean_ref, o_ref):
    x = raw_ref[...] - mean_ref[0:1, :]
    n2 = jnp.sum(x * x, axis=1, keepdims=True)
    o_ref[...] = x * lax.rsqrt(jnp.maximum(n2, 1e-24))


def _finalize(raw, mean_b):
    return pl.pallas_call(
        _finalize_body,
        grid=(BATCH // _FIN_BLK,),
        in_specs=[
            pl.BlockSpec((_FIN_BLK, DIM), lambda i: (i, 0)),
            pl.BlockSpec((8, DIM), lambda i: (0, 0)),
        ],
        out_specs=pl.BlockSpec((_FIN_BLK, DIM), lambda i: (i, 0)),
        out_shape=jax.ShapeDtypeStruct((BATCH, DIM), jnp.float32),
    )(raw, mean_b)


def kernel(indices, embedding):
    idx = indices.astype(jnp.int32)
    table2, acc = _prep(embedding.T)
    q = (idx // _SUM_BLK) * (_SUM_BLK // 2) + (idx & (_SUM_BLK // 2 - 1))
    raw2 = _sc_gather(table2, q)
    mean64 = jnp.sum(acc, axis=1) * (1.0 / NUM_ROWS)
    mean_b = jnp.broadcast_to(mean64[None, :], (8, DIM))
    half = (idx // (_SUM_BLK // 2)) & 1
    raw = jnp.where(half[:, None] == 1, raw2[:, DIM:], raw2[:, :DIM])
    return _finalize(raw, mean_b)


# speedup vs baseline: 1.7437x; 1.1449x over previous
import functools

import jax
import jax.numpy as jnp
from jax import lax
from jax.experimental import pallas as pl
from jax.experimental.pallas import tpu as pltpu
from jax.experimental.pallas import tpu_sc as plsc

NUM_ROWS = 1000000
DIM = 64
BATCH = 16384

_NC = 2
_NS = 16
_NW = _NC * _NS
_B_PER_W = BATCH // _NW
_IDX_CHUNK = 128
_N_CHUNKS = _B_PER_W // _IDX_CHUNK

_SUM_BLK = 16384
_SUM_GRID = (NUM_ROWS + _SUM_BLK - 1) // _SUM_BLK
_SUM_REM = NUM_ROWS - (_SUM_GRID - 1) * _SUM_BLK
_FIN_BLK = 2048


def _sc_gather_body(table_hbm, idx_hbm, out_hbm, idx_v, rows_v, sem):
    wid = lax.axis_index("s") * _NC + lax.axis_index("c")
    base = wid * _B_PER_W
    pltpu.sync_copy(idx_hbm.at[pl.ds(base, _B_PER_W)], idx_v)
    copies = [
        pltpu.async_copy(
            table_hbm.at[idx_v.at[pl.ds(j * _IDX_CHUNK, _IDX_CHUNK)]],
            rows_v.at[pl.ds(j * _IDX_CHUNK, _IDX_CHUNK)],
            sem,
        )
        for j in range(_N_CHUNKS)
    ]
    for c in copies:
        c.wait()
    pltpu.sync_copy(rows_v, out_hbm.at[pl.ds(base, _B_PER_W)])


_sc_gather = pl.kernel(
    _sc_gather_body,
    mesh=plsc.VectorSubcoreMesh(core_axis_name="c", subcore_axis_name="s"),
    compiler_params=pltpu.CompilerParams(use_tc_tiling_on_sc=True),
    out_type=jax.ShapeDtypeStruct((BATCH, 2 * DIM), jnp.float32),
    scratch_types=[
        pltpu.VMEM((_B_PER_W,), jnp.int32),
        pltpu.VMEM((_B_PER_W, 2 * DIM), jnp.float32),
        pltpu.SemaphoreType.DMA,
    ],
)


def _prep_body(x_ref, y_ref, o_ref):
    j = pl.program_id(0)

    @pl.when(j == 0)
    def _init():
        o_ref[...] = jnp.zeros_like(o_ref)

    x = x_ref[...]
    z = jnp.concatenate([x[:, : _SUM_BLK // 2], x[:, _SUM_BLK // 2:]], axis=0)
    ident = jnp.eye(2 * DIM, dtype=jnp.float32)
    y_ref[...] = lax.dot_general(
        z, ident, (((0,), (0,)), ((), ())), preferred_element_type=jnp.float32
    )

    @pl.when(j < _SUM_GRID - 1)
    def _full():
        s = x[:, 0:128]
        for k in range(1, _SUM_BLK // 128):
            s = s + x[:, k * 128:(k + 1) * 128]
        o_ref[...] += s

    @pl.when(j == _SUM_GRID - 1)
    def _tail():
        n_full = _SUM_REM // 128
        s = x[:, 0:128]
        for k in range(1, n_full):
            s = s + x[:, k * 128:(k + 1) * 128]
        part = _SUM_REM - n_full * 128
        if part:
            tail = x[:, n_full * 128:(n_full + 1) * 128]
            lane = lax.broadcasted_iota(jnp.int32, (DIM, 128), 1)
            s = s + jnp.where(lane < part, tail, 0.0)
        o_ref[...] += s


def _prep(table_t):
    return pl.pallas_call(
        _prep_body,
        grid=(_SUM_GRID,),
        in_specs=[pl.BlockSpec((DIM, _SUM_BLK), lambda i: (0, i))],
        out_specs=[
            pl.BlockSpec((_SUM_BLK // 2, 2 * DIM), lambda i: (i, 0)),
            pl.BlockSpec((DIM, 128), lambda i: (0, 0)),
        ],
        out_shape=[
            jax.ShapeDtypeStruct((_SUM_GRID * _SUM_BLK // 2, 2 * DIM), jnp.float32),
            jax.ShapeDtypeStruct((DIM, 128), jnp.float32),
        ],
    )(table_t)


def _finalize_body(raw_ref, mean_ref, o_ref):
    x = raw_ref[...] - mean_ref[0:1, :]
    n2 = jnp.sum(x * x, axis=1, keepdims=True)
    o_ref[...] = x * lax.rsqrt(jnp.maximum(n2, 1e-24))


def _finalize(raw, mean_b):
    return pl.pallas_call(
        _finalize_body,
        grid=(BATCH // _FIN_BLK,),
        in_specs=[
            pl.BlockSpec((_FIN_BLK, DIM), lambda i: (i, 0)),
            pl.BlockSpec((8, DIM), lambda i: (0, 0)),
        ],
        out_specs=pl.BlockSpec((_FIN_BLK, DIM), lambda i: (i, 0)),
        out_shape=jax.ShapeDtypeStruct((BATCH, DIM), jnp.float32),
    )(raw, mean_b)


def kernel(indices, embedding):
    idx = indices.astype(jnp.int32)
    table2, acc = _prep(embedding.T)
    q = (idx // _SUM_BLK) * (_SUM_BLK // 2) + (idx & (_SUM_BLK // 2 - 1))
    raw2 = _sc_gather(table2, q)
    mean64 = jnp.sum(acc, axis=1) * (1.0 / NUM_ROWS)
    mean_b = jnp.broadcast_to(mean64[None, :], (8, DIM))
    half = (idx // (_SUM_BLK // 2)) & 1
    raw = jnp.where(half[:, None] == 1, raw2[:, DIM:], raw2[:, :DIM])
    return _finalize(raw, mean_b)


# speedup vs baseline: 1.7892x; 1.0261x over previous
import functools

import jax
import jax.numpy as jnp
from jax import lax
from jax.experimental import pallas as pl
from jax.experimental.pallas import tpu as pltpu
from jax.experimental.pallas import tpu_sc as plsc

NUM_ROWS = 1000000
DIM = 64
BATCH = 16384

_NC = 2
_NS = 16
_NW = _NC * _NS
_B_PER_W = BATCH // _NW
_IDX_CHUNK = 128
_N_CHUNKS = _B_PER_W // _IDX_CHUNK

_SUM_BLK = 32768
_SUM_GRID = (NUM_ROWS + _SUM_BLK - 1) // _SUM_BLK
_SUM_REM = NUM_ROWS - (_SUM_GRID - 1) * _SUM_BLK
_FIN_BLK = 2048


def _sc_gather_body(table_hbm, idx_hbm, out_hbm, idx_v, rows_v, sem):
    wid = lax.axis_index("s") * _NC + lax.axis_index("c")
    base = wid * _B_PER_W
    pltpu.sync_copy(idx_hbm.at[pl.ds(base, _B_PER_W)], idx_v)
    copies = [
        pltpu.async_copy(
            table_hbm.at[idx_v.at[pl.ds(j * _IDX_CHUNK, _IDX_CHUNK)]],
            rows_v.at[pl.ds(j * _IDX_CHUNK, _IDX_CHUNK)],
            sem,
        )
        for j in range(_N_CHUNKS)
    ]
    for c in copies:
        c.wait()
    pltpu.sync_copy(rows_v, out_hbm.at[pl.ds(base, _B_PER_W)])


_sc_gather = pl.kernel(
    _sc_gather_body,
    mesh=plsc.VectorSubcoreMesh(core_axis_name="c", subcore_axis_name="s"),
    compiler_params=pltpu.CompilerParams(use_tc_tiling_on_sc=True),
    out_type=jax.ShapeDtypeStruct((BATCH, 2 * DIM), jnp.float32),
    scratch_types=[
        pltpu.VMEM((_B_PER_W,), jnp.int32),
        pltpu.VMEM((_B_PER_W, 2 * DIM), jnp.float32),
        pltpu.SemaphoreType.DMA,
    ],
)


def _prep_body(x_ref, y_ref, o_ref):
    j = pl.program_id(0)

    @pl.when(j == 0)
    def _init():
        o_ref[...] = jnp.zeros_like(o_ref)

    x = x_ref[...]
    z = jnp.concatenate([x[:, : _SUM_BLK // 2], x[:, _SUM_BLK // 2:]], axis=0)
    ident = jnp.eye(2 * DIM, dtype=jnp.float32)
    y_ref[...] = lax.dot_general(
        z, ident, (((0,), (0,)), ((), ())), preferred_element_type=jnp.float32
    )

    @pl.when(j < _SUM_GRID - 1)
    def _full():
        s = x[:, 0:128]
        for k in range(1, _SUM_BLK // 128):
            s = s + x[:, k * 128:(k + 1) * 128]
        o_ref[...] += s

    @pl.when(j == _SUM_GRID - 1)
    def _tail():
        n_full = _SUM_REM // 128
        s = x[:, 0:128]
        for k in range(1, n_full):
            s = s + x[:, k * 128:(k + 1) * 128]
        part = _SUM_REM - n_full * 128
        if part:
            tail = x[:, n_full * 128:(n_full + 1) * 128]
            lane = lax.broadcasted_iota(jnp.int32, (DIM, 128), 1)
            s = s + jnp.where(lane < part, tail, 0.0)
        o_ref[...] += s


def _prep(table_t):
    return pl.pallas_call(
        _prep_body,
        grid=(_SUM_GRID,),
        in_specs=[pl.BlockSpec((DIM, _SUM_BLK), lambda i: (0, i))],
        out_specs=[
            pl.BlockSpec((_SUM_BLK // 2, 2 * DIM), lambda i: (i, 0)),
            pl.BlockSpec((DIM, 128), lambda i: (0, 0)),
        ],
        out_shape=[
            jax.ShapeDtypeStruct((_SUM_GRID * _SUM_BLK // 2, 2 * DIM), jnp.float32),
            jax.ShapeDtypeStruct((DIM, 128), jnp.float32),
        ],
    )(table_t)


def _finalize_body(raw_ref, mean_ref, o_ref):
    x = raw_ref[...] - mean_ref[0:1, :]
    n2 = jnp.sum(x * x, axis=1, keepdims=True)
    o_ref[...] = x * lax.rsqrt(jnp.maximum(n2, 1e-24))


def _finalize(raw, mean_b):
    return pl.pallas_call(
        _finalize_body,
        grid=(BATCH // _FIN_BLK,),
        in_specs=[
            pl.BlockSpec((_FIN_BLK, DIM), lambda i: (i, 0)),
            pl.BlockSpec((8, DIM), lambda i: (0, 0)),
        ],
        out_specs=pl.BlockSpec((_FIN_BLK, DIM), lambda i: (i, 0)),
        out_shape=jax.ShapeDtypeStruct((BATCH, DIM), jnp.float32),
    )(raw, mean_b)


def kernel(indices, embedding):
    idx = indices.astype(jnp.int32)
    table2, acc = _prep(embedding.T)
    q = (idx // _SUM_BLK) * (_SUM_BLK // 2) + (idx & (_SUM_BLK // 2 - 1))
    raw2 = _sc_gather(table2, q)
    mean64 = jnp.sum(acc, axis=1) * (1.0 / NUM_ROWS)
    mean_b = jnp.broadcast_to(mean64[None, :], (8, DIM))
    half = (idx // (_SUM_BLK // 2)) & 1
    raw = jnp.where(half[:, None] == 1, raw2[:, DIM:], raw2[:, :DIM])
    return _finalize(raw, mean_b)


# speedup vs baseline: 1.9837x; 1.1087x over previous
import functools

import jax
import jax.numpy as jnp
from jax import lax
from jax.experimental import pallas as pl
from jax.experimental.pallas import tpu as pltpu
from jax.experimental.pallas import tpu_sc as plsc

NUM_ROWS = 1000000
DIM = 64
BATCH = 16384

_NC = 2
_NS = 16
_NW = _NC * _NS
_B_PER_W = BATCH // _NW
_IDX_CHUNK = 128
_N_CHUNKS = _B_PER_W // _IDX_CHUNK

_SUM_BLK = 16384
_SUM_GRID = (NUM_ROWS + _SUM_BLK - 1) // _SUM_BLK
_SUM_REM = NUM_ROWS - (_SUM_GRID - 1) * _SUM_BLK
_FIN_BLK = 2048


def _sc_gather_body(table_hbm, idx_hbm, out_hbm, idx_v, rows_v, sem):
    wid = lax.axis_index("s") * _NC + lax.axis_index("c")
    base = wid * _B_PER_W
    pltpu.sync_copy(idx_hbm.at[pl.ds(base, _B_PER_W)], idx_v)
    copies = [
        pltpu.async_copy(
            table_hbm.at[idx_v.at[pl.ds(j * _IDX_CHUNK, _IDX_CHUNK)]],
            rows_v.at[pl.ds(j * _IDX_CHUNK, _IDX_CHUNK)],
            sem,
        )
        for j in range(_N_CHUNKS)
    ]
    for c in copies:
        c.wait()
    pltpu.sync_copy(rows_v, out_hbm.at[pl.ds(base, _B_PER_W)])


_sc_gather = pl.kernel(
    _sc_gather_body,
    mesh=plsc.VectorSubcoreMesh(core_axis_name="c", subcore_axis_name="s"),
    compiler_params=pltpu.CompilerParams(use_tc_tiling_on_sc=True),
    out_type=jax.ShapeDtypeStruct((BATCH, 2 * DIM), jnp.int32),
    scratch_types=[
        pltpu.VMEM((_B_PER_W,), jnp.int32),
        pltpu.VMEM((_B_PER_W, 2 * DIM), jnp.int32),
        pltpu.SemaphoreType.DMA,
    ],
)


def _prep_body(x_ref, y_ref, o_ref):
    j = pl.program_id(0)

    @pl.when(j == 0)
    def _init():
        o_ref[...] = jnp.zeros_like(o_ref)

    x = x_ref[...]
    qb = _SUM_BLK // 4
    ident = jnp.eye(2 * DIM, dtype=jnp.float32)
    z = jnp.concatenate([x[:, :qb], x[:, qb:2 * qb]], axis=0)
    w = jnp.concatenate([x[:, 2 * qb:3 * qb], x[:, 3 * qb:]], axis=0)
    zt = lax.dot_general(
        z, ident, (((0,), (0,)), ((), ())), preferred_element_type=jnp.float32
    )
    wt = lax.dot_general(
        w, ident, (((0,), (0,)), ((), ())), preferred_element_type=jnp.float32
    )
    uz = lax.bitcast_convert_type(zt, jnp.int32)
    uw = lax.bitcast_convert_type(wt, jnp.int32)
    hi = (uz + 0x8000) & jnp.int32(-65536)
    lo = lax.shift_right_logical(uw + 0x8000, 16)
    y_ref[...] = hi | lo

    @pl.when(j < _SUM_GRID - 1)
    def _full():
        s = x[:, 0:128]
        for k in range(1, _SUM_BLK // 128):
            s = s + x[:, k * 128:(k + 1) * 128]
        o_ref[...] += s

    @pl.when(j == _SUM_GRID - 1)
    def _tail():
        n_full = _SUM_REM // 128
        s = x[:, 0:128]
        for k in range(1, n_full):
            s = s + x[:, k * 128:(k + 1) * 128]
        part = _SUM_REM - n_full * 128
        if part:
            tail = x[:, n_full * 128:(n_full + 1) * 128]
            lane = lax.broadcasted_iota(jnp.int32, (DIM, 128), 1)
            s = s + jnp.where(lane < part, tail, 0.0)
        o_ref[...] += s


def _prep(table_t):
    return pl.pallas_call(
        _prep_body,
        grid=(_SUM_GRID,),
        in_specs=[pl.BlockSpec((DIM, _SUM_BLK), lambda i: (0, i))],
        out_specs=[
            pl.BlockSpec((_SUM_BLK // 4, 2 * DIM), lambda i: (i, 0)),
            pl.BlockSpec((DIM, 128), lambda i: (0, 0)),
        ],
        out_shape=[
            jax.ShapeDtypeStruct((_SUM_GRID * _SUM_BLK // 4, 2 * DIM), jnp.int32),
            jax.ShapeDtypeStruct((DIM, 128), jnp.float32),
        ],
    )(table_t)


def _finalize_body(raw_ref, mean_ref, o_ref):
    x = raw_ref[...] - mean_ref[0:1, :]
    n2 = jnp.sum(x * x, axis=1, keepdims=True)
    o_ref[...] = x * lax.rsqrt(jnp.maximum(n2, 1e-24))


def _finalize(raw, mean_b):
    return pl.pallas_call(
        _finalize_body,
        grid=(BATCH // _FIN_BLK,),
        in_specs=[
            pl.BlockSpec((_FIN_BLK, DIM), lambda i: (i, 0)),
            pl.BlockSpec((8, DIM), lambda i: (0, 0)),
        ],
        out_specs=pl.BlockSpec((_FIN_BLK, DIM), lambda i: (i, 0)),
        out_shape=jax.ShapeDtypeStruct((BATCH, DIM), jnp.float32),
    )(raw, mean_b)


def kernel(indices, embedding):
    idx = indices.astype(jnp.int32)
    table2, acc = _prep(embedding.T)
    qb = _SUM_BLK // 4
    q = (idx // _SUM_BLK) * qb + (idx & (qb - 1))
    raw2 = _sc_gather(table2, q)
    mean64 = jnp.sum(acc, axis=1) * (1.0 / NUM_ROWS)
    mean_b = jnp.broadcast_to(mean64[None, :], (8, DIM))
    k = (idx // qb) & 3
    bits = jnp.where((k & 1)[:, None] == 1, raw2[:, DIM:], raw2[:, :DIM])
    bits = jnp.where(
        (k < 2)[:, None], bits & jnp.int32(-65536), lax.shift_left(bits, 16)
    )
    raw = lax.bitcast_convert_type(bits, jnp.float32)
    return _finalize(raw, mean_b)


# speedup vs baseline: 2.0803x; 1.0487x over previous
import functools

import jax
import jax.numpy as jnp
from jax import lax
from jax.experimental import pallas as pl
from jax.experimental.pallas import tpu as pltpu
from jax.experimental.pallas import tpu_sc as plsc

NUM_ROWS = 1000000
DIM = 64
BATCH = 16384

_NC = 2
_NS = 16
_NW = _NC * _NS
_B_PER_W = BATCH // _NW
_IDX_CHUNK = 128
_N_CHUNKS = _B_PER_W // _IDX_CHUNK

_SUM_BLK = 32768
_SUM_GRID = (NUM_ROWS + _SUM_BLK - 1) // _SUM_BLK
_SUM_REM = NUM_ROWS - (_SUM_GRID - 1) * _SUM_BLK
_FIN_BLK = 2048


def _sc_gather_body(table_hbm, idx_hbm, out_hbm, idx_v, rows_v, sem):
    wid = lax.axis_index("s") * _NC + lax.axis_index("c")
    base = wid * _B_PER_W
    pltpu.sync_copy(idx_hbm.at[pl.ds(base, _B_PER_W)], idx_v)
    copies = [
        pltpu.async_copy(
            table_hbm.at[idx_v.at[pl.ds(j * _IDX_CHUNK, _IDX_CHUNK)]],
            rows_v.at[pl.ds(j * _IDX_CHUNK, _IDX_CHUNK)],
            sem,
        )
        for j in range(_N_CHUNKS)
    ]
    for c in copies:
        c.wait()
    pltpu.sync_copy(rows_v, out_hbm.at[pl.ds(base, _B_PER_W)])


_sc_gather = pl.kernel(
    _sc_gather_body,
    mesh=plsc.VectorSubcoreMesh(core_axis_name="c", subcore_axis_name="s"),
    compiler_params=pltpu.CompilerParams(use_tc_tiling_on_sc=True),
    out_type=jax.ShapeDtypeStruct((BATCH, 2 * DIM), jnp.int32),
    scratch_types=[
        pltpu.VMEM((_B_PER_W,), jnp.int32),
        pltpu.VMEM((_B_PER_W, 2 * DIM), jnp.int32),
        pltpu.SemaphoreType.DMA,
    ],
)


def _prep_body(x_ref, y_ref, o_ref):
    j = pl.program_id(0)

    @pl.when(j == 0)
    def _init():
        o_ref[...] = jnp.zeros_like(o_ref)

    x = x_ref[...]
    qb = _SUM_BLK // 4
    ident = jnp.eye(2 * DIM, dtype=jnp.float32)
    z = jnp.concatenate([x[:, :qb], x[:, qb:2 * qb]], axis=0)
    w = jnp.concatenate([x[:, 2 * qb:3 * qb], x[:, 3 * qb:]], axis=0)
    zt = lax.dot_general(
        z, ident, (((0,), (0,)), ((), ())), preferred_element_type=jnp.float32
    )
    wt = lax.dot_general(
        w, ident, (((0,), (0,)), ((), ())), preferred_element_type=jnp.float32
    )
    uz = lax.bitcast_convert_type(zt, jnp.int32)
    uw = lax.bitcast_convert_type(wt, jnp.int32)
    hi = (uz + 0x8000) & jnp.int32(-65536)
    lo = lax.shift_right_logical(uw + 0x8000, 16)
    y_ref[...] = hi | lo

    @pl.when(j < _SUM_GRID - 1)
    def _full():
        s = x[:, 0:128]
        for k in range(1, _SUM_BLK // 128):
            s = s + x[:, k * 128:(k + 1) * 128]
        o_ref[...] += s

    @pl.when(j == _SUM_GRID - 1)
    def _tail():
        n_full = _SUM_REM // 128
        s = x[:, 0:128]
        for k in range(1, n_full):
            s = s + x[:, k * 128:(k + 1) * 128]
        part = _SUM_REM - n_full * 128
        if part:
            tail = x[:, n_full * 128:(n_full + 1) * 128]
            lane = lax.broadcasted_iota(jnp.int32, (DIM, 128), 1)
            s = s + jnp.where(lane < part, tail, 0.0)
        o_ref[...] += s


def _prep(table_t):
    return pl.pallas_call(
        _prep_body,
        grid=(_SUM_GRID,),
        in_specs=[pl.BlockSpec((DIM, _SUM_BLK), lambda i: (0, i))],
        out_specs=[
            pl.BlockSpec((_SUM_BLK // 4, 2 * DIM), lambda i: (i, 0)),
            pl.BlockSpec((DIM, 128), lambda i: (0, 0)),
        ],
        out_shape=[
            jax.ShapeDtypeStruct((_SUM_GRID * _SUM_BLK // 4, 2 * DIM), jnp.int32),
            jax.ShapeDtypeStruct((DIM, 128), jnp.float32),
        ],
    )(table_t)


def _finalize_body(raw_ref, mean_ref, o_ref):
    x = raw_ref[...] - mean_ref[0:1, :]
    n2 = jnp.sum(x * x, axis=1, keepdims=True)
    o_ref[...] = x * lax.rsqrt(jnp.maximum(n2, 1e-24))


def _finalize(raw, mean_b):
    return pl.pallas_call(
        _finalize_body,
        grid=(BATCH // _FIN_BLK,),
        in_specs=[
            pl.BlockSpec((_FIN_BLK, DIM), lambda i: (i, 0)),
            pl.BlockSpec((8, DIM), lambda i: (0, 0)),
        ],
        out_specs=pl.BlockSpec((_FIN_BLK, DIM), lambda i: (i, 0)),
        out_shape=jax.ShapeDtypeStruct((BATCH, DIM), jnp.float32),
    )(raw, mean_b)


def kernel(indices, embedding):
    idx = indices.astype(jnp.int32)
    table2, acc = _prep(embedding.T)
    qb = _SUM_BLK // 4
    q = (idx // _SUM_BLK) * qb + (idx & (qb - 1))
    raw2 = _sc_gather(table2, q)
    mean64 = jnp.sum(acc, axis=1) * (1.0 / NUM_ROWS)
    mean_b = jnp.broadcast_to(mean64[None, :], (8, DIM))
    k = (idx // qb) & 3
    bits = jnp.where((k & 1)[:, None] == 1, raw2[:, DIM:], raw2[:, :DIM])
    bits = jnp.where(
        (k < 2)[:, None], bits & jnp.int32(-65536), lax.shift_left(bits, 16)
    )
    raw = lax.bitcast_convert_type(bits, jnp.float32)
    return _finalize(raw, mean_b)
